# Initial kernel scaffold; baseline (speedup 1.0000x reference)
import jax
import jax.numpy as jnp
from jax.experimental import pallas as pl


def kernel(x, quantization_mask, Wq, bq, Wk, bk, Wv, bv, Wo, bo, ln1_g, ln1_b, Wtovq, codebook, Wtoemb, lnvq_g, lnvq_b, W1, b1, W2, b2, ln2_g, ln2_b):
    raise NotImplementedError("write your pallas kernel here")


# speedup vs baseline: 1.0599x; 1.0599x over previous
import jax
import jax.numpy as jnp
from jax.experimental import pallas as pl
from jax.experimental.pallas import tpu as pltpu

_T, _C, _H, _D, _FFN, _VQD, _K = 2048, 1024, 16, 64, 4096, 256, 128
_BT = 256
_COMMIT = 1.0


def _ln(y, g, b):
    mu = jnp.mean(y, axis=-1, keepdims=True)
    var = jnp.mean((y - mu) ** 2, axis=-1, keepdims=True)
    return (y - mu) * jax.lax.rsqrt(var + 1e-5) * g + b


def _qkv_kernel(x_ref, w_ref, b_ref, o_ref):
    o_ref[...] = (
        jnp.dot(x_ref[...], w_ref[...], preferred_element_type=jnp.float32)
        + b_ref[...]
    )


def _attn_kernel(q_ref, k_ref, v_ref, o_ref):
    q = q_ref[0]
    k = k_ref[0]
    v = v_ref[0]
    s = jax.lax.dot_general(
        q, k, (((1,), (1,)), ((), ())), preferred_element_type=jnp.float32
    )
    m = jnp.max(s, axis=-1, keepdims=True)
    p = jnp.exp(s - m)
    l = jnp.sum(p, axis=-1, keepdims=True)
    o = jax.lax.dot_general(
        p, v, (((1,), (0,)), ((), ())), preferred_element_type=jnp.float32
    )
    o_ref[0] = o / l


def _pvq_kernel(
    x_ref, a_ref, wo_ref, bo_ref, g1_ref, b1_ref,
    wtv_ref, cb_ref, wte_ref, gv_ref, bv_ref, m_ref,
    x2_ref, ls_ref, ms_ref,
):
    i = pl.program_id(0)
    x = x_ref[...]
    xa = bo_ref[...] + x
    for h in range(_H):
        xa += jnp.dot(
            a_ref[h], wo_ref[h], preferred_element_type=jnp.float32
        )
    x1 = _ln(xa, g1_ref[...], b1_ref[...])
    pf = jnp.dot(x1, wtv_ref[...], preferred_element_type=jnp.float32)
    cb = cb_ref[...]
    d2 = (
        jnp.sum(pf * pf, axis=-1, keepdims=True)
        - 2.0
        * jax.lax.dot_general(
            pf, cb, (((1,), (1,)), ((), ())), preferred_element_type=jnp.float32
        )
        + jnp.sum(cb * cb, axis=-1)[None, :]
    )
    mn = jnp.min(d2, axis=-1, keepdims=True)
    iota = jax.lax.broadcasted_iota(jnp.int32, d2.shape, 1)
    idx = jnp.min(jnp.where(d2 <= mn, iota, _K), axis=-1, keepdims=True)
    onehot = (iota == idx).astype(jnp.float32)
    quant = jnp.dot(onehot, cb, preferred_element_type=jnp.float32)
    mcol = m_ref[...]
    per_tok = jnp.mean((quant - pf) ** 2, axis=-1, keepdims=True)

    @pl.when(i == 0)
    def _():
        ls_ref[...] = jnp.zeros_like(ls_ref)
        ms_ref[...] = jnp.zeros_like(ms_ref)

    ls_ref[...] += jnp.sum(per_tok * mcol).reshape(1, 1)
    ms_ref[...] += jnp.sum(mcol).reshape(1, 1)

    eca = jnp.dot(quant * mcol, wte_ref[...], preferred_element_type=jnp.float32)
    x2 = _ln(x1 + eca, gv_ref[...], bv_ref[...])
    x2_ref[...] = jnp.where(mcol > 0.5, x2, x1)


def _ffn_kernel(x_ref, w1_ref, b1_ref, w2_ref, b2_ref, g2_ref, bl2_ref, o_ref):
    j = pl.program_id(1)
    nj = pl.num_programs(1)
    x = x_ref[...]
    h = jnp.maximum(
        jnp.dot(x, w1_ref[...], preferred_element_type=jnp.float32) + b1_ref[...],
        0.0,
    )
    part = jnp.dot(h, w2_ref[...], preferred_element_type=jnp.float32)

    @pl.when(j == 0)
    def _():
        o_ref[...] = part

    @pl.when(j != 0)
    def _():
        o_ref[...] += part

    @pl.when(j == nj - 1)
    def _():
        y = o_ref[...] + x + b2_ref[...]
        o_ref[...] = _ln(y, g2_ref[...], bl2_ref[...])


def kernel(x, quantization_mask, Wq, bq, Wk, bk, Wv, bv, Wo, bo, ln1_g, ln1_b,
           Wtovq, codebook, Wtoemb, lnvq_g, lnvq_b, W1, b1, W2, b2, ln2_g, ln2_b):
    Tt, Bb, Cc = x.shape
    d = Cc // _H
    s = d ** -0.5
    x2d = x.reshape(Tt, Cc)

    Wqkv = jnp.concatenate([Wq * s, Wk, Wv], axis=1)
    bqkv = jnp.concatenate([bq * s, bk, bv]).reshape(1, 3 * Cc)

    nt = Tt // _BT
    qkv = pl.pallas_call(
        _qkv_kernel,
        grid=(nt, 6),
        in_specs=[
            pl.BlockSpec((_BT, Cc), lambda i, j: (i, 0)),
            pl.BlockSpec((Cc, Cc // 2), lambda i, j: (0, j)),
            pl.BlockSpec((1, Cc // 2), lambda i, j: (0, j)),
        ],
        out_specs=pl.BlockSpec((_BT, Cc // 2), lambda i, j: (i, j)),
        out_shape=jax.ShapeDtypeStruct((Tt, 3 * Cc), jnp.float32),
        compiler_params=pltpu.CompilerParams(
            dimension_semantics=("parallel", "parallel")
        ),
    )(x2d, Wqkv, bqkv)

    qkv_h = qkv.reshape(Tt, 3 * _H, _D).transpose(1, 0, 2)
    attn_out = pl.pallas_call(
        _attn_kernel,
        grid=(_H, nt),
        in_specs=[
            pl.BlockSpec((1, _BT, _D), lambda h, i: (h, i, 0)),
            pl.BlockSpec((1, Tt, _D), lambda h, i: (_H + h, 0, 0)),
            pl.BlockSpec((1, Tt, _D), lambda h, i: (2 * _H + h, 0, 0)),
        ],
        out_specs=pl.BlockSpec((1, _BT, _D), lambda h, i: (h, i, 0)),
        out_shape=jax.ShapeDtypeStruct((_H, Tt, _D), jnp.float32),
        compiler_params=pltpu.CompilerParams(
            dimension_semantics=("parallel", "parallel")
        ),
    )(qkv_h, qkv_h, qkv_h)

    mflat = quantization_mask.reshape(Tt, 1).astype(jnp.float32)
    row = lambda v: v.reshape(1, -1)
    x2, ls, ms = pl.pallas_call(
        _pvq_kernel,
        grid=(nt,),
        in_specs=[
            pl.BlockSpec((_BT, Cc), lambda i: (i, 0)),
            pl.BlockSpec((_H, _BT, _D), lambda i: (0, i, 0)),
            pl.BlockSpec((_H, _D, Cc), lambda i: (0, 0, 0)),
            pl.BlockSpec((1, Cc), lambda i: (0, 0)),
            pl.BlockSpec((1, Cc), lambda i: (0, 0)),
            pl.BlockSpec((1, Cc), lambda i: (0, 0)),
            pl.BlockSpec((Cc, _VQD), lambda i: (0, 0)),
            pl.BlockSpec((_K, _VQD), lambda i: (0, 0)),
            pl.BlockSpec((_VQD, Cc), lambda i: (0, 0)),
            pl.BlockSpec((1, Cc), lambda i: (0, 0)),
            pl.BlockSpec((1, Cc), lambda i: (0, 0)),
            pl.BlockSpec((_BT, 1), lambda i: (i, 0)),
        ],
        out_specs=[
            pl.BlockSpec((_BT, Cc), lambda i: (i, 0)),
            pl.BlockSpec((1, 1), lambda i: (0, 0)),
            pl.BlockSpec((1, 1), lambda i: (0, 0)),
        ],
        out_shape=[
            jax.ShapeDtypeStruct((Tt, Cc), jnp.float32),
            jax.ShapeDtypeStruct((1, 1), jnp.float32),
            jax.ShapeDtypeStruct((1, 1), jnp.float32),
        ],
        compiler_params=pltpu.CompilerParams(
            dimension_semantics=("arbitrary",)
        ),
    )(x2d, attn_out, Wo.reshape(_H, _D, Cc), row(bo), row(ln1_g), row(ln1_b),
      Wtovq, codebook, Wtoemb, row(lnvq_g), row(lnvq_b), mflat)

    commit_loss = (_COMMIT * ls[0, 0] / jnp.maximum(ms[0, 0], 1.0)).astype(
        jnp.float32
    )

    nf = 4
    bf = _FFN // nf
    x3 = pl.pallas_call(
        _ffn_kernel,
        grid=(nt, nf),
        in_specs=[
            pl.BlockSpec((_BT, Cc), lambda i, j: (i, 0)),
            pl.BlockSpec((Cc, bf), lambda i, j: (0, j)),
            pl.BlockSpec((1, bf), lambda i, j: (0, j)),
            pl.BlockSpec((bf, Cc), lambda i, j: (j, 0)),
            pl.BlockSpec((1, Cc), lambda i, j: (0, 0)),
            pl.BlockSpec((1, Cc), lambda i, j: (0, 0)),
            pl.BlockSpec((1, Cc), lambda i, j: (0, 0)),
        ],
        out_specs=pl.BlockSpec((_BT, Cc), lambda i, j: (i, 0)),
        out_shape=jax.ShapeDtypeStruct((Tt, Cc), jnp.float32),
        compiler_params=pltpu.CompilerParams(
            dimension_semantics=("parallel", "arbitrary")
        ),
    )(x2, W1, row(b1), W2, row(b2), row(ln2_g), row(ln2_b))

    return x3.reshape(Tt, Bb, Cc), commit_loss


# speedup vs baseline: 1.1713x; 1.1051x over previous
import jax
import jax.numpy as jnp
from jax.experimental import pallas as pl
from jax.experimental.pallas import tpu as pltpu

_T, _C, _H, _D, _FFN, _VQD, _K = 2048, 1024, 16, 64, 4096, 256, 128
_BT = 256
_COMMIT = 1.0


def _ln(y, g, b):
    mu = jnp.mean(y, axis=-1, keepdims=True)
    var = jnp.mean((y - mu) ** 2, axis=-1, keepdims=True)
    return (y - mu) * jax.lax.rsqrt(var + 1e-5) * g + b


def _bdot(a, b):
    return jax.lax.dot_general(
        a.astype(jnp.bfloat16), b.astype(jnp.bfloat16),
        (((1,), (0,)), ((), ())), preferred_element_type=jnp.float32,
    )


def _qkv_kernel(x_ref, w_ref, b_ref, o_ref):
    o_ref[...] = _bdot(x_ref[...], w_ref[...]) + b_ref[...]


def _attn_kernel(q_ref, k_ref, v_ref, o_ref):
    q = q_ref[0].astype(jnp.bfloat16)
    k = k_ref[0].astype(jnp.bfloat16)
    v = v_ref[0].astype(jnp.bfloat16)
    s = jax.lax.dot_general(
        q, k, (((1,), (1,)), ((), ())), preferred_element_type=jnp.float32
    )
    m = jnp.max(s, axis=-1, keepdims=True)
    p = jnp.exp(s - m)
    l = jnp.sum(p, axis=-1, keepdims=True)
    o = jax.lax.dot_general(
        p.astype(jnp.bfloat16), v, (((1,), (0,)), ((), ())),
        preferred_element_type=jnp.float32,
    )
    o_ref[0] = o / l


def _pvq_kernel(
    x_ref, a_ref, wo_ref, bo_ref, g1_ref, b1_ref,
    wtv_ref, cb_ref, wte_ref, gv_ref, bv_ref, m_ref,
    x2_ref, ls_ref, ms_ref,
):
    i = pl.program_id(0)
    x = x_ref[...]
    xa = bo_ref[...] + x
    for h in range(_H):
        xa += _bdot(a_ref[h], wo_ref[h])
    x1 = _ln(xa, g1_ref[...], b1_ref[...])
    pf = jnp.dot(x1, wtv_ref[...], preferred_element_type=jnp.float32)
    cb = cb_ref[...]
    d2 = (
        jnp.sum(pf * pf, axis=-1, keepdims=True)
        - 2.0
        * jax.lax.dot_general(
            pf, cb, (((1,), (1,)), ((), ())), preferred_element_type=jnp.float32
        )
        + jnp.sum(cb * cb, axis=-1)[None, :]
    )
    mn = jnp.min(d2, axis=-1, keepdims=True)
    iota = jax.lax.broadcasted_iota(jnp.int32, d2.shape, 1)
    idx = jnp.min(jnp.where(d2 <= mn, iota, _K), axis=-1, keepdims=True)
    onehot = (iota == idx).astype(jnp.float32)
    quant = jnp.dot(onehot, cb, preferred_element_type=jnp.float32)
    mcol = m_ref[...]
    per_tok = jnp.mean((quant - pf) ** 2, axis=-1, keepdims=True)

    @pl.when(i == 0)
    def _():
        ls_ref[...] = jnp.zeros_like(ls_ref)
        ms_ref[...] = jnp.zeros_like(ms_ref)

    ls_ref[...] += jnp.sum(per_tok * mcol).reshape(1, 1)
    ms_ref[...] += jnp.sum(mcol).reshape(1, 1)

    eca = _bdot(quant * mcol, wte_ref[...])
    x2 = _ln(x1 + eca, gv_ref[...], bv_ref[...])
    x2_ref[...] = jnp.where(mcol > 0.5, x2, x1)


def _ffn_kernel(x_ref, w1_ref, b1_ref, w2_ref, b2_ref, g2_ref, bl2_ref, o_ref):
    j = pl.program_id(1)
    nj = pl.num_programs(1)
    x = x_ref[...]
    h = jnp.maximum(_bdot(x, w1_ref[...]) + b1_ref[...], 0.0)
    part = _bdot(h, w2_ref[...])

    @pl.when(j == 0)
    def _():
        o_ref[...] = part

    @pl.when(j != 0)
    def _():
        o_ref[...] += part

    @pl.when(j == nj - 1)
    def _():
        y = o_ref[...] + x + b2_ref[...]
        o_ref[...] = _ln(y, g2_ref[...], bl2_ref[...])


def kernel(x, quantization_mask, Wq, bq, Wk, bk, Wv, bv, Wo, bo, ln1_g, ln1_b,
           Wtovq, codebook, Wtoemb, lnvq_g, lnvq_b, W1, b1, W2, b2, ln2_g, ln2_b):
    Tt, Bb, Cc = x.shape
    d = Cc // _H
    s = d ** -0.5
    x2d = x.reshape(Tt, Cc)

    Wqkv = jnp.concatenate([Wq * s, Wk, Wv], axis=1)
    bqkv = jnp.concatenate([bq * s, bk, bv]).reshape(1, 3 * Cc)

    nt = Tt // _BT
    qkv = pl.pallas_call(
        _qkv_kernel,
        grid=(nt, 6),
        in_specs=[
            pl.BlockSpec((_BT, Cc), lambda i, j: (i, 0)),
            pl.BlockSpec((Cc, Cc // 2), lambda i, j: (0, j)),
            pl.BlockSpec((1, Cc // 2), lambda i, j: (0, j)),
        ],
        out_specs=pl.BlockSpec((_BT, Cc // 2), lambda i, j: (i, j)),
        out_shape=jax.ShapeDtypeStruct((Tt, 3 * Cc), jnp.float32),
        compiler_params=pltpu.CompilerParams(
            dimension_semantics=("parallel", "parallel")
        ),
    )(x2d, Wqkv, bqkv)

    qkv_h = qkv.reshape(Tt, 3 * _H, _D).transpose(1, 0, 2)
    attn_out = pl.pallas_call(
        _attn_kernel,
        grid=(_H, nt),
        in_specs=[
            pl.BlockSpec((1, _BT, _D), lambda h, i: (h, i, 0)),
            pl.BlockSpec((1, Tt, _D), lambda h, i: (_H + h, 0, 0)),
            pl.BlockSpec((1, Tt, _D), lambda h, i: (2 * _H + h, 0, 0)),
        ],
        out_specs=pl.BlockSpec((1, _BT, _D), lambda h, i: (h, i, 0)),
        out_shape=jax.ShapeDtypeStruct((_H, Tt, _D), jnp.float32),
        compiler_params=pltpu.CompilerParams(
            dimension_semantics=("parallel", "parallel")
        ),
    )(qkv_h, qkv_h, qkv_h)

    mflat = quantization_mask.reshape(Tt, 1).astype(jnp.float32)
    row = lambda v: v.reshape(1, -1)
    x2, ls, ms = pl.pallas_call(
        _pvq_kernel,
        grid=(nt,),
        in_specs=[
            pl.BlockSpec((_BT, Cc), lambda i: (i, 0)),
            pl.BlockSpec((_H, _BT, _D), lambda i: (0, i, 0)),
            pl.BlockSpec((_H, _D, Cc), lambda i: (0, 0, 0)),
            pl.BlockSpec((1, Cc), lambda i: (0, 0)),
            pl.BlockSpec((1, Cc), lambda i: (0, 0)),
            pl.BlockSpec((1, Cc), lambda i: (0, 0)),
            pl.BlockSpec((Cc, _VQD), lambda i: (0, 0)),
            pl.BlockSpec((_K, _VQD), lambda i: (0, 0)),
            pl.BlockSpec((_VQD, Cc), lambda i: (0, 0)),
            pl.BlockSpec((1, Cc), lambda i: (0, 0)),
            pl.BlockSpec((1, Cc), lambda i: (0, 0)),
            pl.BlockSpec((_BT, 1), lambda i: (i, 0)),
        ],
        out_specs=[
            pl.BlockSpec((_BT, Cc), lambda i: (i, 0)),
            pl.BlockSpec((1, 1), lambda i: (0, 0)),
            pl.BlockSpec((1, 1), lambda i: (0, 0)),
        ],
        out_shape=[
            jax.ShapeDtypeStruct((Tt, Cc), jnp.float32),
            jax.ShapeDtypeStruct((1, 1), jnp.float32),
            jax.ShapeDtypeStruct((1, 1), jnp.float32),
        ],
        compiler_params=pltpu.CompilerParams(
            dimension_semantics=("arbitrary",)
        ),
    )(x2d, attn_out, Wo.reshape(_H, _D, Cc), row(bo), row(ln1_g), row(ln1_b),
      Wtovq, codebook, Wtoemb, row(lnvq_g), row(lnvq_b), mflat)

    commit_loss = (_COMMIT * ls[0, 0] / jnp.maximum(ms[0, 0], 1.0)).astype(
        jnp.float32
    )

    nf = 4
    bf = _FFN // nf
    x3 = pl.pallas_call(
        _ffn_kernel,
        grid=(nt, nf),
        in_specs=[
            pl.BlockSpec((_BT, Cc), lambda i, j: (i, 0)),
            pl.BlockSpec((Cc, bf), lambda i, j: (0, j)),
            pl.BlockSpec((1, bf), lambda i, j: (0, j)),
            pl.BlockSpec((bf, Cc), lambda i, j: (j, 0)),
            pl.BlockSpec((1, Cc), lambda i, j: (0, 0)),
            pl.BlockSpec((1, Cc), lambda i, j: (0, 0)),
            pl.BlockSpec((1, Cc), lambda i, j: (0, 0)),
        ],
        out_specs=pl.BlockSpec((_BT, Cc), lambda i, j: (i, 0)),
        out_shape=jax.ShapeDtypeStruct((Tt, Cc), jnp.float32),
        compiler_params=pltpu.CompilerParams(
            dimension_semantics=("parallel", "arbitrary")
        ),
    )(x2, W1, row(b1), W2, row(b2), row(ln2_g), row(ln2_b))

    return x3.reshape(Tt, Bb, Cc), commit_loss


# speedup vs baseline: 1.3714x; 1.1709x over previous
import jax
import jax.numpy as jnp
from jax.experimental import pallas as pl
from jax.experimental.pallas import tpu as pltpu

_T, _C, _H, _D, _FFN, _VQD, _K = 2048, 1024, 16, 64, 4096, 256, 128
_BT = 256
_COMMIT = 1.0


def _ln(y, g, b):
    mu = jnp.mean(y, axis=-1, keepdims=True)
    var = jnp.mean((y - mu) ** 2, axis=-1, keepdims=True)
    return (y - mu) * jax.lax.rsqrt(var + 1e-5) * g + b


def _bdot(a, b):
    return jax.lax.dot_general(
        a.astype(jnp.bfloat16), b.astype(jnp.bfloat16),
        (((1,), (0,)), ((), ())), preferred_element_type=jnp.float32,
    )


def _qkv_kernel(x_ref, w_ref, b_ref, o_ref):
    o_ref[...] = (_bdot(x_ref[...], w_ref[...]) + b_ref[...]).astype(jnp.bfloat16)


def _attn_kernel(q_ref, k_ref, v_ref, o_ref):
    for h in range(_H):
        q = q_ref[h]
        k = k_ref[h]
        v = v_ref[h]
        s = jax.lax.dot_general(
            q, k, (((1,), (1,)), ((), ())), preferred_element_type=jnp.float32
        )
        m = jnp.max(s, axis=-1, keepdims=True)
        p = jnp.exp(s - m)
        l = jnp.sum(p, axis=-1, keepdims=True)
        o = jax.lax.dot_general(
            p.astype(jnp.bfloat16), v, (((1,), (0,)), ((), ())),
            preferred_element_type=jnp.float32,
        )
        o_ref[h] = (o / l).astype(jnp.bfloat16)


def _pvq_kernel(
    x_ref, a_ref, wo_ref, bo_ref, g1_ref, b1_ref,
    wtv_ref, cb_ref, wte_ref, gv_ref, bv_ref, m_ref,
    x2_ref, ls_ref, ms_ref,
):
    i = pl.program_id(0)
    x = x_ref[...]
    xa = bo_ref[...] + x
    for h in range(_H):
        xa += _bdot(a_ref[h], wo_ref[h])
    x1 = _ln(xa, g1_ref[...], b1_ref[...])
    pf = jnp.dot(x1, wtv_ref[...], preferred_element_type=jnp.float32)
    cb = cb_ref[...]
    d2 = (
        jnp.sum(pf * pf, axis=-1, keepdims=True)
        - 2.0
        * jax.lax.dot_general(
            pf, cb, (((1,), (1,)), ((), ())), preferred_element_type=jnp.float32
        )
        + jnp.sum(cb * cb, axis=-1)[None, :]
    )
    mn = jnp.min(d2, axis=-1, keepdims=True)
    iota = jax.lax.broadcasted_iota(jnp.int32, d2.shape, 1)
    idx = jnp.min(jnp.where(d2 <= mn, iota, _K), axis=-1, keepdims=True)
    onehot = (iota == idx).astype(jnp.float32)
    quant = jnp.dot(onehot, cb, preferred_element_type=jnp.float32)
    mcol = m_ref[...]
    per_tok = jnp.mean((quant - pf) ** 2, axis=-1, keepdims=True)

    @pl.when(i == 0)
    def _():
        ls_ref[...] = jnp.zeros_like(ls_ref)
        ms_ref[...] = jnp.zeros_like(ms_ref)

    ls_ref[...] += jnp.sum(per_tok * mcol).reshape(1, 1)
    ms_ref[...] += jnp.sum(mcol).reshape(1, 1)

    eca = _bdot(quant * mcol, wte_ref[...])
    x2 = _ln(x1 + eca, gv_ref[...], bv_ref[...])
    x2_ref[...] = jnp.where(mcol > 0.5, x2, x1)


def _ffn_kernel(x_ref, w1_ref, b1_ref, w2_ref, b2_ref, g2_ref, bl2_ref, o_ref):
    j = pl.program_id(1)
    nj = pl.num_programs(1)
    x = x_ref[...]
    h = jnp.maximum(_bdot(x, w1_ref[...]) + b1_ref[...], 0.0)
    part = _bdot(h, w2_ref[...])

    @pl.when(j == 0)
    def _():
        o_ref[...] = part

    @pl.when(j != 0)
    def _():
        o_ref[...] += part

    @pl.when(j == nj - 1)
    def _():
        y = o_ref[...] + x + b2_ref[...]
        o_ref[...] = _ln(y, g2_ref[...], bl2_ref[...])


def kernel(x, quantization_mask, Wq, bq, Wk, bk, Wv, bv, Wo, bo, ln1_g, ln1_b,
           Wtovq, codebook, Wtoemb, lnvq_g, lnvq_b, W1, b1, W2, b2, ln2_g, ln2_b):
    Tt, Bb, Cc = x.shape
    d = Cc // _H
    s = d ** -0.5
    x2d = x.reshape(Tt, Cc)

    Wqkv = jnp.concatenate([Wq * s, Wk, Wv], axis=1)
    bqkv = jnp.concatenate([bq * s, bk, bv]).reshape(1, 3 * Cc)

    nt = Tt // _BT
    qkv = pl.pallas_call(
        _qkv_kernel,
        grid=(nt, 6),
        in_specs=[
            pl.BlockSpec((_BT, Cc), lambda i, j: (i, 0)),
            pl.BlockSpec((Cc, Cc // 2), lambda i, j: (0, j)),
            pl.BlockSpec((1, Cc // 2), lambda i, j: (0, j)),
        ],
        out_specs=pl.BlockSpec((_BT, Cc // 2), lambda i, j: (i, j)),
        out_shape=jax.ShapeDtypeStruct((Tt, 3 * Cc), jnp.bfloat16),
        compiler_params=pltpu.CompilerParams(
            dimension_semantics=("parallel", "parallel")
        ),
    )(x2d, Wqkv, bqkv)

    qkv_h = qkv.reshape(Tt, 3 * _H, _D).transpose(1, 0, 2)
    attn_out = pl.pallas_call(
        _attn_kernel,
        grid=(nt,),
        in_specs=[
            pl.BlockSpec((_H, _BT, _D), lambda i: (0, i, 0)),
            pl.BlockSpec((_H, Tt, _D), lambda i: (1, 0, 0)),
            pl.BlockSpec((_H, Tt, _D), lambda i: (2, 0, 0)),
        ],
        out_specs=pl.BlockSpec((_H, _BT, _D), lambda i: (0, i, 0)),
        out_shape=jax.ShapeDtypeStruct((_H, Tt, _D), jnp.bfloat16),
        compiler_params=pltpu.CompilerParams(
            dimension_semantics=("parallel",)
        ),
    )(qkv_h, qkv_h, qkv_h)

    mflat = quantization_mask.reshape(Tt, 1).astype(jnp.float32)
    row = lambda v: v.reshape(1, -1)
    x2, ls, ms = pl.pallas_call(
        _pvq_kernel,
        grid=(nt,),
        in_specs=[
            pl.BlockSpec((_BT, Cc), lambda i: (i, 0)),
            pl.BlockSpec((_H, _BT, _D), lambda i: (0, i, 0)),
            pl.BlockSpec((_H, _D, Cc), lambda i: (0, 0, 0)),
            pl.BlockSpec((1, Cc), lambda i: (0, 0)),
            pl.BlockSpec((1, Cc), lambda i: (0, 0)),
            pl.BlockSpec((1, Cc), lambda i: (0, 0)),
            pl.BlockSpec((Cc, _VQD), lambda i: (0, 0)),
            pl.BlockSpec((_K, _VQD), lambda i: (0, 0)),
            pl.BlockSpec((_VQD, Cc), lambda i: (0, 0)),
            pl.BlockSpec((1, Cc), lambda i: (0, 0)),
            pl.BlockSpec((1, Cc), lambda i: (0, 0)),
            pl.BlockSpec((_BT, 1), lambda i: (i, 0)),
        ],
        out_specs=[
            pl.BlockSpec((_BT, Cc), lambda i: (i, 0)),
            pl.BlockSpec((1, 1), lambda i: (0, 0)),
            pl.BlockSpec((1, 1), lambda i: (0, 0)),
        ],
        out_shape=[
            jax.ShapeDtypeStruct((Tt, Cc), jnp.float32),
            jax.ShapeDtypeStruct((1, 1), jnp.float32),
            jax.ShapeDtypeStruct((1, 1), jnp.float32),
        ],
        compiler_params=pltpu.CompilerParams(
            dimension_semantics=("arbitrary",)
        ),
    )(x2d, attn_out, Wo.reshape(_H, _D, Cc), row(bo), row(ln1_g), row(ln1_b),
      Wtovq, codebook, Wtoemb, row(lnvq_g), row(lnvq_b), mflat)

    commit_loss = (_COMMIT * ls[0, 0] / jnp.maximum(ms[0, 0], 1.0)).astype(
        jnp.float32
    )

    nf = 4
    bf = _FFN // nf
    x3 = pl.pallas_call(
        _ffn_kernel,
        grid=(nt, nf),
        in_specs=[
            pl.BlockSpec((_BT, Cc), lambda i, j: (i, 0)),
            pl.BlockSpec((Cc, bf), lambda i, j: (0, j)),
            pl.BlockSpec((1, bf), lambda i, j: (0, j)),
            pl.BlockSpec((bf, Cc), lambda i, j: (j, 0)),
            pl.BlockSpec((1, Cc), lambda i, j: (0, 0)),
            pl.BlockSpec((1, Cc), lambda i, j: (0, 0)),
            pl.BlockSpec((1, Cc), lambda i, j: (0, 0)),
        ],
        out_specs=pl.BlockSpec((_BT, Cc), lambda i, j: (i, 0)),
        out_shape=jax.ShapeDtypeStruct((Tt, Cc), jnp.float32),
        compiler_params=pltpu.CompilerParams(
            dimension_semantics=("parallel", "arbitrary")
        ),
    )(x2, W1, row(b1), W2, row(b2), row(ln2_g), row(ln2_b))

    return x3.reshape(Tt, Bb, Cc), commit_loss


# speedup vs baseline: 1.6859x; 1.2293x over previous
import jax
import jax.numpy as jnp
from jax.experimental import pallas as pl
from jax.experimental.pallas import tpu as pltpu

_T, _C, _H, _D, _FFN, _VQD, _K = 2048, 1024, 16, 64, 4096, 256, 128
_BT = 256
_COMMIT = 1.0
_BF = jnp.bfloat16


def _ln(y, g, b):
    mu = jnp.mean(y, axis=-1, keepdims=True)
    var = jnp.mean((y - mu) ** 2, axis=-1, keepdims=True)
    return (y - mu) * jax.lax.rsqrt(var + 1e-5) * g + b


def _bdot(a, b):
    return jax.lax.dot_general(
        a.astype(_BF), b.astype(_BF),
        (((1,), (0,)), ((), ())), preferred_element_type=jnp.float32,
    )


def _qkv_kernel(x_ref, wq_ref, wk_ref, wv_ref, bq_ref, bk_ref, bv_ref, o_ref):
    x = x_ref[...].astype(_BF)
    scale = _D ** -0.5
    q = (_bdot(x, wq_ref[...]) + bq_ref[...]) * scale
    k = _bdot(x, wk_ref[...]) + bk_ref[...]
    v = _bdot(x, wv_ref[...]) + bv_ref[...]
    o_ref[:, 0:_C] = q.astype(_BF)
    o_ref[:, _C:2 * _C] = k.astype(_BF)
    o_ref[:, 2 * _C:3 * _C] = v.astype(_BF)


def _attn_kernel(q_ref, k_ref, v_ref, o_ref):
    for h in range(_H):
        q = q_ref[h]
        k = k_ref[h]
        v = v_ref[h]
        s = jax.lax.dot_general(
            q, k, (((1,), (1,)), ((), ())), preferred_element_type=jnp.float32
        )
        m = jnp.max(s, axis=-1, keepdims=True)
        p = jnp.exp(s - m)
        l = jnp.sum(p, axis=-1, keepdims=True)
        o = jax.lax.dot_general(
            p.astype(_BF), v, (((1,), (0,)), ((), ())),
            preferred_element_type=jnp.float32,
        )
        o_ref[h] = (o / l).astype(_BF)


def _pvq_kernel(
    x_ref, a_ref, wo_ref, bo_ref, g1_ref, b1_ref,
    wtv_ref, cb_ref, wte_ref, gv_ref, bv_ref, m_ref,
    x2_ref, ls_ref, ms_ref,
):
    i = pl.program_id(0)
    x = x_ref[...]
    xa = bo_ref[...] + x
    for h in range(_H):
        xa += _bdot(a_ref[h], wo_ref[h])
    x1 = _ln(xa, g1_ref[...], b1_ref[...])
    pf = jnp.dot(x1, wtv_ref[...], preferred_element_type=jnp.float32)
    cb = cb_ref[...]
    d2 = (
        jnp.sum(pf * pf, axis=-1, keepdims=True)
        - 2.0
        * jax.lax.dot_general(
            pf, cb, (((1,), (1,)), ((), ())), preferred_element_type=jnp.float32
        )
        + jnp.sum(cb * cb, axis=-1)[None, :]
    )
    mn = jnp.min(d2, axis=-1, keepdims=True)
    iota = jax.lax.broadcasted_iota(jnp.int32, d2.shape, 1)
    idx = jnp.min(jnp.where(d2 <= mn, iota, _K), axis=-1, keepdims=True)
    onehot = (iota == idx).astype(jnp.float32)
    quant = jnp.dot(onehot, cb, preferred_element_type=jnp.float32)
    mcol = m_ref[...]
    per_tok = jnp.mean((quant - pf) ** 2, axis=-1, keepdims=True)

    @pl.when(i == 0)
    def _():
        ls_ref[...] = jnp.zeros_like(ls_ref)
        ms_ref[...] = jnp.zeros_like(ms_ref)

    ls_ref[...] += jnp.sum(per_tok * mcol).reshape(1, 1)
    ms_ref[...] += jnp.sum(mcol).reshape(1, 1)

    eca = _bdot(quant * mcol, wte_ref[...])
    x2 = _ln(x1 + eca, gv_ref[...], bv_ref[...])
    x2_ref[...] = jnp.where(mcol > 0.5, x2, x1)


def _ffn_kernel(x_ref, w1_ref, b1_ref, w2_ref, b2_ref, g2_ref, bl2_ref, o_ref):
    x = x_ref[...]
    xb = x.astype(_BF)
    acc = x + b2_ref[...]
    nj = _FFN // _C
    for j in range(nj):
        w1j = w1_ref[:, j * _C:(j + 1) * _C]
        h = jnp.maximum(
            jax.lax.dot_general(
                xb, w1j, (((1,), (0,)), ((), ())),
                preferred_element_type=jnp.float32,
            )
            + b1_ref[:, j * _C:(j + 1) * _C],
            0.0,
        )
        w2j = w2_ref[j * _C:(j + 1) * _C, :]
        acc += jax.lax.dot_general(
            h.astype(_BF), w2j, (((1,), (0,)), ((), ())),
            preferred_element_type=jnp.float32,
        )
    o_ref[...] = _ln(acc, g2_ref[...], bl2_ref[...])


def kernel(x, quantization_mask, Wq, bq, Wk, bk, Wv, bv, Wo, bo, ln1_g, ln1_b,
           Wtovq, codebook, Wtoemb, lnvq_g, lnvq_b, W1, b1, W2, b2, ln2_g, ln2_b):
    Tt, Bb, Cc = x.shape
    x2d = x.reshape(Tt, Cc)
    nt = Tt // _BT
    row = lambda v: v.reshape(1, -1)
    bf = lambda a: a.astype(_BF)

    qkv = pl.pallas_call(
        _qkv_kernel,
        grid=(nt,),
        in_specs=[
            pl.BlockSpec((_BT, Cc), lambda i: (i, 0)),
            pl.BlockSpec((Cc, Cc), lambda i: (0, 0)),
            pl.BlockSpec((Cc, Cc), lambda i: (0, 0)),
            pl.BlockSpec((Cc, Cc), lambda i: (0, 0)),
            pl.BlockSpec((1, Cc), lambda i: (0, 0)),
            pl.BlockSpec((1, Cc), lambda i: (0, 0)),
            pl.BlockSpec((1, Cc), lambda i: (0, 0)),
        ],
        out_specs=pl.BlockSpec((_BT, 3 * Cc), lambda i: (i, 0)),
        out_shape=jax.ShapeDtypeStruct((Tt, 3 * Cc), _BF),
        compiler_params=pltpu.CompilerParams(
            dimension_semantics=("parallel",)
        ),
    )(x2d, bf(Wq), bf(Wk), bf(Wv), row(bq), row(bk), row(bv))

    qkv_h = qkv.reshape(Tt, 3 * _H, _D).transpose(1, 0, 2)
    attn_out = pl.pallas_call(
        _attn_kernel,
        grid=(nt,),
        in_specs=[
            pl.BlockSpec((_H, _BT, _D), lambda i: (0, i, 0)),
            pl.BlockSpec((_H, Tt, _D), lambda i: (1, 0, 0)),
            pl.BlockSpec((_H, Tt, _D), lambda i: (2, 0, 0)),
        ],
        out_specs=pl.BlockSpec((_H, _BT, _D), lambda i: (0, i, 0)),
        out_shape=jax.ShapeDtypeStruct((_H, Tt, _D), _BF),
        compiler_params=pltpu.CompilerParams(
            dimension_semantics=("parallel",)
        ),
    )(qkv_h, qkv_h, qkv_h)

    mflat = quantization_mask.reshape(Tt, 1).astype(jnp.float32)
    x2, ls, ms = pl.pallas_call(
        _pvq_kernel,
        grid=(nt,),
        in_specs=[
            pl.BlockSpec((_BT, Cc), lambda i: (i, 0)),
            pl.BlockSpec((_H, _BT, _D), lambda i: (0, i, 0)),
            pl.BlockSpec((_H, _D, Cc), lambda i: (0, 0, 0)),
            pl.BlockSpec((1, Cc), lambda i: (0, 0)),
            pl.BlockSpec((1, Cc), lambda i: (0, 0)),
            pl.BlockSpec((1, Cc), lambda i: (0, 0)),
            pl.BlockSpec((Cc, _VQD), lambda i: (0, 0)),
            pl.BlockSpec((_K, _VQD), lambda i: (0, 0)),
            pl.BlockSpec((_VQD, Cc), lambda i: (0, 0)),
            pl.BlockSpec((1, Cc), lambda i: (0, 0)),
            pl.BlockSpec((1, Cc), lambda i: (0, 0)),
            pl.BlockSpec((_BT, 1), lambda i: (i, 0)),
        ],
        out_specs=[
            pl.BlockSpec((_BT, Cc), lambda i: (i, 0)),
            pl.BlockSpec((1, 1), lambda i: (0, 0)),
            pl.BlockSpec((1, 1), lambda i: (0, 0)),
        ],
        out_shape=[
            jax.ShapeDtypeStruct((Tt, Cc), jnp.float32),
            jax.ShapeDtypeStruct((1, 1), jnp.float32),
            jax.ShapeDtypeStruct((1, 1), jnp.float32),
        ],
        compiler_params=pltpu.CompilerParams(
            dimension_semantics=("arbitrary",)
        ),
    )(x2d, attn_out, bf(Wo).reshape(_H, _D, Cc), row(bo), row(ln1_g),
      row(ln1_b), Wtovq, codebook, bf(Wtoemb), row(lnvq_g), row(lnvq_b),
      mflat)

    commit_loss = (_COMMIT * ls[0, 0] / jnp.maximum(ms[0, 0], 1.0)).astype(
        jnp.float32
    )

    x3 = pl.pallas_call(
        _ffn_kernel,
        grid=(nt,),
        in_specs=[
            pl.BlockSpec((_BT, Cc), lambda i: (i, 0)),
            pl.BlockSpec((Cc, _FFN), lambda i: (0, 0)),
            pl.BlockSpec((1, _FFN), lambda i: (0, 0)),
            pl.BlockSpec((_FFN, Cc), lambda i: (0, 0)),
            pl.BlockSpec((1, Cc), lambda i: (0, 0)),
            pl.BlockSpec((1, Cc), lambda i: (0, 0)),
            pl.BlockSpec((1, Cc), lambda i: (0, 0)),
        ],
        out_specs=pl.BlockSpec((_BT, Cc), lambda i: (i, 0)),
        out_shape=jax.ShapeDtypeStruct((Tt, Cc), jnp.float32),
        compiler_params=pltpu.CompilerParams(
            dimension_semantics=("parallel",)
        ),
    )(x2, bf(W1), row(b1), bf(W2), row(b2), row(ln2_g), row(ln2_b))

    return x3.reshape(Tt, Bb, Cc), commit_loss


# speedup vs baseline: 1.9003x; 1.1272x over previous
import jax
import jax.numpy as jnp
from jax.experimental import pallas as pl
from jax.experimental.pallas import tpu as pltpu

_T, _C, _H, _D, _FFN, _VQD, _K = 2048, 1024, 16, 64, 4096, 256, 128
_BT = 256
_COMMIT = 1.0
_BF = jnp.bfloat16


def _ln(y, g, b):
    mu = jnp.mean(y, axis=-1, keepdims=True)
    var = jnp.mean((y - mu) ** 2, axis=-1, keepdims=True)
    return (y - mu) * jax.lax.rsqrt(var + 1e-5) * g + b


def _bdot(a, b):
    return jax.lax.dot_general(
        a.astype(_BF), b.astype(_BF),
        (((1,), (0,)), ((), ())), preferred_element_type=jnp.float32,
    )


def _qkv_kernel(x_ref, wq_ref, wk_ref, wv_ref, bq_ref, bk_ref, bv_ref, o_ref):
    x = x_ref[...].astype(_BF)
    scale = _D ** -0.5
    q = (_bdot(x, wq_ref[...]) + bq_ref[...]) * scale
    k = _bdot(x, wk_ref[...]) + bk_ref[...]
    v = _bdot(x, wv_ref[...]) + bv_ref[...]
    o_ref[:, 0:_C] = q.astype(_BF)
    o_ref[:, _C:2 * _C] = k.astype(_BF)
    o_ref[:, 2 * _C:3 * _C] = v.astype(_BF)


def _attn_kernel(q_ref, k_ref, v_ref, o_ref, kh_s, vh_s):
    i = pl.program_id(0)

    @pl.when(i == 0)
    def _():
        for h in range(_H):
            kh_s[h] = k_ref[:, h * _D:(h + 1) * _D]
            vh_s[h] = v_ref[:, h * _D:(h + 1) * _D]

    for h in range(_H):
        q = q_ref[:, h * _D:(h + 1) * _D]
        k = kh_s[h]
        v = vh_s[h]
        s = jax.lax.dot_general(
            q, k, (((1,), (1,)), ((), ())), preferred_element_type=jnp.float32
        )
        m = jnp.max(s, axis=-1, keepdims=True)
        p = jnp.exp(s - m)
        l = jnp.sum(p, axis=-1, keepdims=True)
        o = jax.lax.dot_general(
            p.astype(_BF), v, (((1,), (0,)), ((), ())),
            preferred_element_type=jnp.float32,
        )
        o_ref[:, h * _D:(h + 1) * _D] = (o / l).astype(_BF)


def _pvq_kernel(
    x_ref, a_ref, wo_ref, bo_ref, g1_ref, b1_ref,
    wtv_ref, cb_ref, wte_ref, gv_ref, bv_ref, m_ref,
    x2_ref, ls_ref, ms_ref,
):
    i = pl.program_id(0)
    x = x_ref[...]
    xa = _bdot(a_ref[...], wo_ref[...]) + bo_ref[...] + x
    x1 = _ln(xa, g1_ref[...], b1_ref[...])
    pf = jnp.dot(x1, wtv_ref[...], preferred_element_type=jnp.float32)
    cb = cb_ref[...]
    d2 = (
        jnp.sum(pf * pf, axis=-1, keepdims=True)
        - 2.0
        * jax.lax.dot_general(
            pf, cb, (((1,), (1,)), ((), ())), preferred_element_type=jnp.float32
        )
        + jnp.sum(cb * cb, axis=-1)[None, :]
    )
    mn = jnp.min(d2, axis=-1, keepdims=True)
    iota = jax.lax.broadcasted_iota(jnp.int32, d2.shape, 1)
    idx = jnp.min(jnp.where(d2 <= mn, iota, _K), axis=-1, keepdims=True)
    onehot = (iota == idx).astype(jnp.float32)
    quant = jnp.dot(onehot, cb, preferred_element_type=jnp.float32)
    mcol = m_ref[...]
    per_tok = jnp.mean((quant - pf) ** 2, axis=-1, keepdims=True)

    @pl.when(i == 0)
    def _():
        ls_ref[...] = jnp.zeros_like(ls_ref)
        ms_ref[...] = jnp.zeros_like(ms_ref)

    ls_ref[...] += jnp.sum(per_tok * mcol).reshape(1, 1)
    ms_ref[...] += jnp.sum(mcol).reshape(1, 1)

    eca = _bdot(quant * mcol, wte_ref[...])
    x2 = _ln(x1 + eca, gv_ref[...], bv_ref[...])
    x2_ref[...] = jnp.where(mcol > 0.5, x2, x1)


def _ffn_kernel(x_ref, w1_ref, b1_ref, w2_ref, b2_ref, g2_ref, bl2_ref, o_ref):
    x = x_ref[...]
    xb = x.astype(_BF)
    acc = x + b2_ref[...]
    nj = _FFN // _C
    for j in range(nj):
        w1j = w1_ref[:, j * _C:(j + 1) * _C]
        h = jnp.maximum(
            jax.lax.dot_general(
                xb, w1j, (((1,), (0,)), ((), ())),
                preferred_element_type=jnp.float32,
            )
            + b1_ref[:, j * _C:(j + 1) * _C],
            0.0,
        )
        w2j = w2_ref[j * _C:(j + 1) * _C, :]
        acc += jax.lax.dot_general(
            h.astype(_BF), w2j, (((1,), (0,)), ((), ())),
            preferred_element_type=jnp.float32,
        )
    o_ref[...] = _ln(acc, g2_ref[...], bl2_ref[...])


def kernel(x, quantization_mask, Wq, bq, Wk, bk, Wv, bv, Wo, bo, ln1_g, ln1_b,
           Wtovq, codebook, Wtoemb, lnvq_g, lnvq_b, W1, b1, W2, b2, ln2_g, ln2_b):
    Tt, Bb, Cc = x.shape
    x2d = x.reshape(Tt, Cc)
    nt = Tt // _BT
    row = lambda v: v.reshape(1, -1)
    bf = lambda a: a.astype(_BF)

    qkv = pl.pallas_call(
        _qkv_kernel,
        grid=(nt,),
        in_specs=[
            pl.BlockSpec((_BT, Cc), lambda i: (i, 0)),
            pl.BlockSpec((Cc, Cc), lambda i: (0, 0)),
            pl.BlockSpec((Cc, Cc), lambda i: (0, 0)),
            pl.BlockSpec((Cc, Cc), lambda i: (0, 0)),
            pl.BlockSpec((1, Cc), lambda i: (0, 0)),
            pl.BlockSpec((1, Cc), lambda i: (0, 0)),
            pl.BlockSpec((1, Cc), lambda i: (0, 0)),
        ],
        out_specs=pl.BlockSpec((_BT, 3 * Cc), lambda i: (i, 0)),
        out_shape=jax.ShapeDtypeStruct((Tt, 3 * Cc), _BF),
        compiler_params=pltpu.CompilerParams(
            dimension_semantics=("parallel",)
        ),
    )(x2d, bf(Wq), bf(Wk), bf(Wv), row(bq), row(bk), row(bv))

    attn_out = pl.pallas_call(
        _attn_kernel,
        grid=(nt,),
        in_specs=[
            pl.BlockSpec((_BT, Cc), lambda i: (i, 0)),
            pl.BlockSpec((Tt, Cc), lambda i: (0, 1)),
            pl.BlockSpec((Tt, Cc), lambda i: (0, 2)),
        ],
        out_specs=pl.BlockSpec((_BT, Cc), lambda i: (i, 0)),
        out_shape=jax.ShapeDtypeStruct((Tt, Cc), _BF),
        scratch_shapes=[
            pltpu.VMEM((_H, Tt, _D), _BF),
            pltpu.VMEM((_H, Tt, _D), _BF),
        ],
        compiler_params=pltpu.CompilerParams(
            dimension_semantics=("arbitrary",)
        ),
    )(qkv, qkv, qkv)

    mflat = quantization_mask.reshape(Tt, 1).astype(jnp.float32)
    x2, ls, ms = pl.pallas_call(
        _pvq_kernel,
        grid=(nt,),
        in_specs=[
            pl.BlockSpec((_BT, Cc), lambda i: (i, 0)),
            pl.BlockSpec((_BT, Cc), lambda i: (i, 0)),
            pl.BlockSpec((Cc, Cc), lambda i: (0, 0)),
            pl.BlockSpec((1, Cc), lambda i: (0, 0)),
            pl.BlockSpec((1, Cc), lambda i: (0, 0)),
            pl.BlockSpec((1, Cc), lambda i: (0, 0)),
            pl.BlockSpec((Cc, _VQD), lambda i: (0, 0)),
            pl.BlockSpec((_K, _VQD), lambda i: (0, 0)),
            pl.BlockSpec((_VQD, Cc), lambda i: (0, 0)),
            pl.BlockSpec((1, Cc), lambda i: (0, 0)),
            pl.BlockSpec((1, Cc), lambda i: (0, 0)),
            pl.BlockSpec((_BT, 1), lambda i: (i, 0)),
        ],
        out_specs=[
            pl.BlockSpec((_BT, Cc), lambda i: (i, 0)),
            pl.BlockSpec((1, 1), lambda i: (0, 0)),
            pl.BlockSpec((1, 1), lambda i: (0, 0)),
        ],
        out_shape=[
            jax.ShapeDtypeStruct((Tt, Cc), jnp.float32),
            jax.ShapeDtypeStruct((1, 1), jnp.float32),
            jax.ShapeDtypeStruct((1, 1), jnp.float32),
        ],
        compiler_params=pltpu.CompilerParams(
            dimension_semantics=("arbitrary",)
        ),
    )(x2d, attn_out, bf(Wo), row(bo), row(ln1_g),
      row(ln1_b), Wtovq, codebook, bf(Wtoemb), row(lnvq_g), row(lnvq_b),
      mflat)

    commit_loss = (_COMMIT * ls[0, 0] / jnp.maximum(ms[0, 0], 1.0)).astype(
        jnp.float32
    )

    x3 = pl.pallas_call(
        _ffn_kernel,
        grid=(nt,),
        in_specs=[
            pl.BlockSpec((_BT, Cc), lambda i: (i, 0)),
            pl.BlockSpec((Cc, _FFN), lambda i: (0, 0)),
            pl.BlockSpec((1, _FFN), lambda i: (0, 0)),
            pl.BlockSpec((_FFN, Cc), lambda i: (0, 0)),
            pl.BlockSpec((1, Cc), lambda i: (0, 0)),
            pl.BlockSpec((1, Cc), lambda i: (0, 0)),
            pl.BlockSpec((1, Cc), lambda i: (0, 0)),
        ],
        out_specs=pl.BlockSpec((_BT, Cc), lambda i: (i, 0)),
        out_shape=jax.ShapeDtypeStruct((Tt, Cc), jnp.float32),
        compiler_params=pltpu.CompilerParams(
            dimension_semantics=("parallel",)
        ),
    )(x2, bf(W1), row(b1), bf(W2), row(b2), row(ln2_g), row(ln2_b))

    return x3.reshape(Tt, Bb, Cc), commit_loss
